# baseline (speedup 1.0000x reference)
import functools

import jax
import jax.numpy as jnp
from jax import lax
from jax.experimental import pallas as pl
from jax.experimental.pallas import tpu as pltpu

MASK_RATIO = 0.75
ROWS_PER_BLOCK = 32

_ROT_EVEN = (13, 15, 26, 6)
_ROT_ODD = (17, 29, 16, 24)
_KS = (0, 42, 42 ^ 0x1BD11BDA)


def _rotl(x, r):
    return lax.shift_left(x, jnp.int32(r)) | lax.shift_right_logical(
        x, jnp.int32(32 - r)
    )


def _threefry_bits(x0, x1):
    x1 = x1 + jnp.int32(_KS[1])
    first = True
    for g in range(5):
        rots = _ROT_EVEN if g % 2 == 0 else _ROT_ODD
        for r in rots:
            x0 = x1 if first else x0 + x1
            first = False
            x1 = _rotl(x1, r)
            x1 = x1 ^ x0
        x0 = x0 + jnp.int32(_KS[(g + 1) % 3])
        x1 = x1 + jnp.int32((_KS[(g + 2) % 3] + g + 1) & 0xFFFFFFFF)
    return x0 ^ x1


def _mask_body(fm_ref, out_ref, *, rows, num_tokens, num_visible):
    g = pl.program_id(0)
    shape = (rows, num_tokens)
    row = lax.broadcasted_iota(jnp.int32, shape, 0)
    col = lax.broadcasted_iota(jnp.int32, shape, 1)
    base = g * jnp.int32(rows * num_tokens)
    p = base + row * jnp.int32(num_tokens) + col
    bits = _threefry_bits(jnp.zeros(shape, jnp.int32), p)
    u = lax.shift_right_logical(bits, jnp.int32(9))

    nvis = jnp.int32(num_visible)

    def value_step(_, carry):
        lo, hi = carry
        mid = lax.shift_right_logical(lo + hi, jnp.int32(1))
        cnt = jnp.sum((u <= mid).astype(jnp.int32), axis=1, keepdims=True)
        pred = cnt >= nvis
        return jnp.where(pred, lo, mid + 1), jnp.where(pred, mid, hi)

    lo0 = jnp.zeros((rows, 1), jnp.int32)
    hi0 = jnp.full((rows, 1), (1 << 23) - 1, jnp.int32)
    t_val, _ = lax.fori_loop(0, 23, value_step, (lo0, hi0), unroll=True)

    c_less = jnp.sum((u < t_val).astype(jnp.int32), axis=1, keepdims=True)
    r_need = nvis - c_less
    eq_t = u == t_val

    chunks = num_tokens // 128
    eqb = eq_t.astype(jnp.int32).astype(jnp.bfloat16)
    x2d = eqb.reshape(rows * chunks, 128)
    tri128 = (lax.broadcasted_iota(jnp.int32, (128, 128), 0)
              < lax.broadcasted_iota(jnp.int32, (128, 128), 1)
              ).astype(jnp.bfloat16)
    within = lax.dot_general(x2d, tri128, (((1,), (0,)), ((), ())),
                             preferred_element_type=jnp.float32)
    tot = (within[:, 127:128] + x2d[:, 127:128].astype(jnp.float32)
           ).reshape(rows, chunks)
    tri_c = (lax.broadcasted_iota(jnp.int32, (chunks, chunks), 0)
             < lax.broadcasted_iota(jnp.int32, (chunks, chunks), 1)
             ).astype(jnp.bfloat16)
    chunk_pre = lax.dot_general(tot.astype(jnp.bfloat16), tri_c,
                                (((1,), (0,)), ((), ())),
                                preferred_element_type=jnp.float32)
    prefix = (chunk_pre.reshape(rows, chunks, 1)
              + within.reshape(rows, chunks, 128)).reshape(rows, num_tokens)
    tie_vis = eq_t & (prefix < r_need.astype(jnp.float32))

    visible = (u < t_val) | tie_vis
    fm = fm_ref[0]
    out_ref[:, :] = jnp.where(fm != jnp.int32(0),
                              visible.astype(jnp.float32),
                              jnp.float32(1.0))


def kernel(x, force_mask):
    batch_size, num_tokens = x.shape
    ones_mask = jnp.ones((batch_size, num_tokens), dtype=jnp.float32)
    if num_tokens <= 1:
        return ones_mask
    num_masked = int(round(num_tokens * MASK_RATIO))
    num_masked = min(max(1, num_masked), num_tokens - 1)
    num_visible = num_tokens - num_masked

    rows = ROWS_PER_BLOCK if batch_size % ROWS_PER_BLOCK == 0 else 1
    grid = batch_size // rows
    fm = jnp.asarray(force_mask, jnp.int32).reshape((1,))

    body = functools.partial(
        _mask_body, rows=rows, num_tokens=num_tokens, num_visible=num_visible
    )
    return pl.pallas_call(
        body,
        grid=(grid,),
        in_specs=[pl.BlockSpec(memory_space=pltpu.SMEM)],
        out_specs=pl.BlockSpec((rows, num_tokens), lambda i: (i, 0)),
        out_shape=jax.ShapeDtypeStruct((batch_size, num_tokens), jnp.float32),
        compiler_params=pltpu.CompilerParams(
            dimension_semantics=("parallel",),
        ),
    )(fm)


# speedup vs baseline: 1.2025x; 1.2025x over previous
import functools

import jax
import jax.numpy as jnp
from jax import lax
from jax.experimental import pallas as pl
from jax.experimental.pallas import tpu as pltpu

MASK_RATIO = 0.75
ROWS_PER_BLOCK = 32

_ROT_EVEN = (13, 15, 26, 6)
_ROT_ODD = (17, 29, 16, 24)
_KS = (0, 42, 42 ^ 0x1BD11BDA)


def _rotl(x, r):
    return lax.shift_left(x, jnp.int32(r)) | lax.shift_right_logical(
        x, jnp.int32(32 - r)
    )


def _threefry_bits(x0, x1):
    x1 = x1 + jnp.int32(_KS[1])
    first = True
    for g in range(5):
        rots = _ROT_EVEN if g % 2 == 0 else _ROT_ODD
        for r in rots:
            x0 = x1 if first else x0 + x1
            first = False
            x1 = _rotl(x1, r)
            x1 = x1 ^ x0
        x0 = x0 + jnp.int32(_KS[(g + 1) % 3])
        x1 = x1 + jnp.int32((_KS[(g + 2) % 3] + g + 1) & 0xFFFFFFFF)
    return x0 ^ x1


def _mask_body(fm_ref, out_ref, *, rows, num_tokens, num_visible):
    g = pl.program_id(0)
    shape = (rows, num_tokens)
    row = lax.broadcasted_iota(jnp.int32, shape, 0)
    col = lax.broadcasted_iota(jnp.int32, shape, 1)
    base = g * jnp.int32(rows * num_tokens)
    p = base + row * jnp.int32(num_tokens) + col
    bits = _threefry_bits(jnp.zeros(shape, jnp.int32), p)
    u = lax.shift_right_logical(bits, jnp.int32(9))

    nvis = jnp.int32(num_visible)

    def value_step(_, carry):
        lo, hi = carry
        mid = lax.shift_right_logical(lo + hi, jnp.int32(1))
        neg = jnp.sum(lax.shift_right_arithmetic(u - (mid + 1), jnp.int32(31)),
                      axis=1, keepdims=True)
        pred = neg <= -nvis
        return jnp.where(pred, lo, mid + 1), jnp.where(pred, mid, hi)

    lo0 = jnp.zeros((rows, 1), jnp.int32)
    hi0 = jnp.full((rows, 1), (1 << 23) - 1, jnp.int32)
    t_val, _ = lax.fori_loop(0, 23, value_step, (lo0, hi0))

    c_less = -jnp.sum(lax.shift_right_arithmetic(u - t_val, jnp.int32(31)),
                      axis=1, keepdims=True)
    r_need = nvis - c_less
    eq_t = u == t_val

    chunks = num_tokens // 128
    eqb = eq_t.astype(jnp.int32).astype(jnp.bfloat16)
    x2d = eqb.reshape(rows * chunks, 128)
    tri128 = (lax.broadcasted_iota(jnp.int32, (128, 128), 0)
              < lax.broadcasted_iota(jnp.int32, (128, 128), 1)
              ).astype(jnp.bfloat16)
    within = lax.dot_general(x2d, tri128, (((1,), (0,)), ((), ())),
                             preferred_element_type=jnp.float32)
    tot = (within[:, 127:128] + x2d[:, 127:128].astype(jnp.float32)
           ).reshape(rows, chunks)
    tri_c = (lax.broadcasted_iota(jnp.int32, (chunks, chunks), 0)
             < lax.broadcasted_iota(jnp.int32, (chunks, chunks), 1)
             ).astype(jnp.bfloat16)
    chunk_pre = lax.dot_general(tot.astype(jnp.bfloat16), tri_c,
                                (((1,), (0,)), ((), ())),
                                preferred_element_type=jnp.float32)
    prefix = (chunk_pre.reshape(rows, chunks, 1)
              + within.reshape(rows, chunks, 128)).reshape(rows, num_tokens)
    tie_vis = eq_t & (prefix < r_need.astype(jnp.float32))

    visible = (u < t_val) | tie_vis
    fm = fm_ref[0]
    out_ref[:, :] = jnp.where(fm != jnp.int32(0),
                              visible.astype(jnp.float32),
                              jnp.float32(1.0))


def kernel(x, force_mask):
    batch_size, num_tokens = x.shape
    ones_mask = jnp.ones((batch_size, num_tokens), dtype=jnp.float32)
    if num_tokens <= 1:
        return ones_mask
    num_masked = int(round(num_tokens * MASK_RATIO))
    num_masked = min(max(1, num_masked), num_tokens - 1)
    num_visible = num_tokens - num_masked

    rows = ROWS_PER_BLOCK if batch_size % ROWS_PER_BLOCK == 0 else 1
    grid = batch_size // rows
    fm = jnp.asarray(force_mask, jnp.int32).reshape((1,))

    body = functools.partial(
        _mask_body, rows=rows, num_tokens=num_tokens, num_visible=num_visible
    )
    return pl.pallas_call(
        body,
        grid=(grid,),
        in_specs=[pl.BlockSpec(memory_space=pltpu.SMEM)],
        out_specs=pl.BlockSpec((rows, num_tokens), lambda i: (i, 0)),
        out_shape=jax.ShapeDtypeStruct((batch_size, num_tokens), jnp.float32),
        compiler_params=pltpu.CompilerParams(
            dimension_semantics=("parallel",),
        ),
    )(fm)


# speedup vs baseline: 1.9406x; 1.6138x over previous
import functools

import jax
import jax.numpy as jnp
from jax import lax
from jax.experimental import pallas as pl
from jax.experimental.pallas import tpu as pltpu

MASK_RATIO = 0.75
ROWS_PER_BLOCK = 32

_ROT_EVEN = (13, 15, 26, 6)
_ROT_ODD = (17, 29, 16, 24)
_KS = (0, 42, 42 ^ 0x1BD11BDA)


def _rotl(x, r):
    return lax.shift_left(x, jnp.int32(r)) | lax.shift_right_logical(
        x, jnp.int32(32 - r)
    )


def _threefry_bits(x0, x1):
    x1 = x1 + jnp.int32(_KS[1])
    first = True
    for g in range(5):
        rots = _ROT_EVEN if g % 2 == 0 else _ROT_ODD
        for r in rots:
            x0 = x1 if first else x0 + x1
            first = False
            x1 = _rotl(x1, r)
            x1 = x1 ^ x0
        x0 = x0 + jnp.int32(_KS[(g + 1) % 3])
        x1 = x1 + jnp.int32((_KS[(g + 2) % 3] + g + 1) & 0xFFFFFFFF)
    return x0 ^ x1


def _mask_body(fm_ref, base_ref, out_ref, *, rows, num_tokens, num_visible):
    g = pl.program_id(0)
    shape = (rows, num_tokens)
    row = lax.broadcasted_iota(jnp.int32, shape, 0)
    col = lax.broadcasted_iota(jnp.int32, shape, 1)
    base = (base_ref[0] + g * jnp.int32(rows)) * jnp.int32(num_tokens)
    p = base + row * jnp.int32(num_tokens) + col
    bits = _threefry_bits(jnp.zeros(shape, jnp.int32), p)
    u = lax.shift_right_logical(bits, jnp.int32(9))

    nvis = jnp.int32(num_visible)

    def value_step(_, carry):
        lo, hi = carry
        mid = lax.shift_right_logical(lo + hi, jnp.int32(1))
        neg = jnp.sum(lax.shift_right_arithmetic(u - (mid + 1), jnp.int32(31)),
                      axis=1, keepdims=True)
        pred = neg <= -nvis
        return jnp.where(pred, lo, mid + 1), jnp.where(pred, mid, hi)

    lo0 = jnp.zeros((rows, 1), jnp.int32)
    hi0 = jnp.full((rows, 1), (1 << 23) - 1, jnp.int32)
    t_val, _ = lax.fori_loop(0, 23, value_step, (lo0, hi0))

    c_less = -jnp.sum(lax.shift_right_arithmetic(u - t_val, jnp.int32(31)),
                      axis=1, keepdims=True)
    r_need = nvis - c_less
    eq_t = u == t_val

    chunks = num_tokens // 128
    eqb = eq_t.astype(jnp.int32).astype(jnp.bfloat16)
    x2d = eqb.reshape(rows * chunks, 128)
    tri128 = (lax.broadcasted_iota(jnp.int32, (128, 128), 0)
              < lax.broadcasted_iota(jnp.int32, (128, 128), 1)
              ).astype(jnp.bfloat16)
    within = lax.dot_general(x2d, tri128, (((1,), (0,)), ((), ())),
                             preferred_element_type=jnp.float32)
    tot = (within[:, 127:128] + x2d[:, 127:128].astype(jnp.float32)
           ).reshape(rows, chunks)
    tri_c = (lax.broadcasted_iota(jnp.int32, (chunks, chunks), 0)
             < lax.broadcasted_iota(jnp.int32, (chunks, chunks), 1)
             ).astype(jnp.bfloat16)
    chunk_pre = lax.dot_general(tot.astype(jnp.bfloat16), tri_c,
                                (((1,), (0,)), ((), ())),
                                preferred_element_type=jnp.float32)
    prefix = (chunk_pre.reshape(rows, chunks, 1)
              + within.reshape(rows, chunks, 128)).reshape(rows, num_tokens)
    tie_vis = eq_t & (prefix < r_need.astype(jnp.float32))

    visible = (u < t_val) | tie_vis
    fm = fm_ref[0]
    out_ref[:, :] = jnp.where(fm != jnp.int32(0),
                              visible.astype(jnp.float32),
                              jnp.float32(1.0))


def _masker_call(fm, base, shard_batch, num_tokens, num_visible):
    rows = ROWS_PER_BLOCK if shard_batch % ROWS_PER_BLOCK == 0 else 1
    grid = shard_batch // rows
    body = functools.partial(
        _mask_body, rows=rows, num_tokens=num_tokens, num_visible=num_visible
    )
    return pl.pallas_call(
        body,
        grid=(grid,),
        in_specs=[pl.BlockSpec(memory_space=pltpu.SMEM),
                  pl.BlockSpec(memory_space=pltpu.SMEM)],
        out_specs=pl.BlockSpec((rows, num_tokens), lambda i: (i, 0)),
        out_shape=jax.ShapeDtypeStruct((shard_batch, num_tokens), jnp.float32),
        compiler_params=pltpu.CompilerParams(
            dimension_semantics=("parallel",),
        ),
    )(fm, base)


def kernel(x, force_mask):
    batch_size, num_tokens = x.shape
    ones_mask = jnp.ones((batch_size, num_tokens), dtype=jnp.float32)
    if num_tokens <= 1:
        return ones_mask
    num_masked = int(round(num_tokens * MASK_RATIO))
    num_masked = min(max(1, num_masked), num_tokens - 1)
    num_visible = num_tokens - num_masked

    fm = jnp.asarray(force_mask, jnp.int32).reshape((1,))

    ndev = len(jax.devices())
    if ndev > 1 and batch_size % (ndev * ROWS_PER_BLOCK) == 0:
        mesh = jax.make_mesh((ndev,), ("b",))
        spec = jax.sharding.PartitionSpec
        shard_batch = batch_size // ndev

        def shard_fn(fm_arr):
            sid = lax.axis_index("b")
            base = (sid.astype(jnp.int32) * jnp.int32(shard_batch)
                    ).reshape((1,))
            return _masker_call(fm_arr, base, shard_batch, num_tokens,
                                num_visible)

        return jax.shard_map(shard_fn, mesh=mesh, in_specs=spec(),
                             out_specs=spec("b", None), check_vma=False)(fm)

    base0 = jnp.zeros((1,), jnp.int32)
    return _masker_call(fm, base0, batch_size, num_tokens, num_visible)
